# baseline (speedup 1.0000x reference)
import functools

import jax
import jax.numpy as jnp
import numpy as np
from jax import lax
from jax.experimental import pallas as pl
from jax.experimental.pallas import tpu as pltpu
from jax.experimental.pallas import tpu_sc as plsc

N = 10000
D_IN = 128
HEADS = 8
HID = 16
DH = HEADS * HID
DHALF = DH // 2
D_OUT = 128

NP_ = 10112
NC = 2
NS = 16
NW = NC * NS
K = 128
ZR = NP_ // NS

_mesh = plsc.VectorSubcoreMesh(
    core_axis_name="c", subcore_axis_name="s", num_cores=NC, num_subcores=NS
)
_sc_params = pltpu.CompilerParams(use_tc_tiling_on_sc=False)

f32 = jnp.float32
i32 = jnp.int32


def _k0_body(x_ref, wg_ref, a1_ref, a2_ref, hh_ref, als_ref, ald_ref, m_ref):
    h = jnp.dot(x_ref[...], wg_ref[...], preferred_element_type=f32)
    hh_ref[0] = h[:, :DHALF]
    hh_ref[1] = h[:, DHALF:]
    als = jnp.dot(h, a1_ref[...], preferred_element_type=f32)
    ald = jnp.dot(h, a2_ref[...], preferred_element_type=f32)
    als_ref[...] = als
    ald_ref[...] = ald
    s = jnp.max(als, axis=0, keepdims=True) + jnp.max(ald, axis=0, keepdims=True)
    mlr = jnp.maximum(s, 0.2 * s)
    col = lax.broadcasted_iota(i32, (1, 16), 1)
    m_ref[...] = jnp.where(col < HEADS, mlr, 1e9)


def _kd_body(acc_ref, den_ref, wgcn_ref, r1_ref, r2_ref, out_ref):
    acc = jnp.concatenate([acc_ref[0], acc_ref[1]], axis=1)
    den_all = den_ref[0] + den_ref[1]
    den128 = jnp.dot(den_all, r1_ref[...], preferred_element_type=f32)
    deg128 = jnp.dot(den_all, r2_ref[...], preferred_element_type=f32)
    ygat = acc / (den128 + 1e-16)
    dinv = jnp.where(deg128 > 0, lax.rsqrt(deg128), 0.0)
    h2p = dinv * jnp.dot(ygat, wgcn_ref[...], preferred_element_type=f32)
    out_ref[0] = h2p[:, :DHALF]
    out_ref[1] = h2p[:, DHALF:]


def _ke_body(acc_ref, den_ref, r2_ref, out_ref):
    den_all = den_ref[0] + den_ref[1]
    deg128 = jnp.dot(den_all, r2_ref[...], preferred_element_type=f32)
    dinv = jnp.where(deg128 > 0, lax.rsqrt(deg128), 0.0)
    out_ref[...] = dinv * jnp.concatenate([acc_ref[0], acc_ref[1]], axis=1)


def _make_attn_kernel(chw):
    @functools.partial(
        pl.kernel,
        out_type=(
            jax.ShapeDtypeStruct((NW * chw, K, 16), f32),
            jax.ShapeDtypeStruct((NC, NP_, 16), f32),
        ),
        mesh=_mesh,
        compiler_params=_sc_params,
        scratch_types=(
            pltpu.VMEM((2, K), i32),
            pltpu.VMEM((2, K), i32),
            pltpu.VMEM((2, K), i32),
            pltpu.VMEM((2, K), i32),
            pltpu.SemaphoreType.DMA,
            pltpu.SemaphoreType.DMA,
            pltpu.SemaphoreType.DMA,
            pltpu.SemaphoreType.DMA,
            pltpu.VMEM((K, 16), f32),
            pltpu.VMEM((K, 16), f32),
            pltpu.VMEM((K, 16), f32),
            pltpu.VMEM((K, 16), f32),
            pltpu.VMEM((K, 16), f32),
            pltpu.VMEM((K, 16), f32),
            pltpu.VMEM((1, 16), f32),
            pltpu.VMEM((ZR, 16), f32),
            pltpu.SemaphoreType.DMA,
            pltpu.SemaphoreType.DMA,
            pltpu.SemaphoreType.DMA,
            pltpu.SemaphoreType.DMA,
            pltpu.SemaphoreType.DMA,
            pltpu.SemaphoreType.DMA,
            pltpu.SemaphoreType.DMA,
            pltpu.SemaphoreType.DMA,
            pltpu.VMEM_SHARED((NP_, 16), f32),
        ),
    )
    def attn(als_hbm, ald_hbm, m_hbm, sd_hbm, ex_hbm, den_hbm,
             ix2a, ix2b, ix2c, ix2d, sixa, sixb, sixc, sixd,
             g1a, g1b, g2a, g2b, exa, exb2, mv, zb,
             sg1a, sg1b, sg2a, sg2b, sexa, sexb, ssca, sscb, den_sh):
        c = lax.axis_index("c")
        s = lax.axis_index("s")
        wid = s * NC + c
        ix2 = (ix2a, ix2b, ix2c, ix2d)
        six = (sixa, sixb, sixc, sixd)
        g1 = (g1a, g1b)
        g2 = (g2a, g2b)
        exv = (exa, exb2)
        sg1 = (sg1a, sg1b)
        sg2 = (sg2a, sg2b)
        sex = (sexa, sexb)
        ssc = (ssca, sscb)
        zvec = jnp.zeros((16,), f32)

        def zrow(i, _):
            zb[i] = zvec
            return 0

        lax.fori_loop(0, ZR, zrow, 0)
        pltpu.sync_copy(zb, den_sh.at[pl.ds(s * ZR, ZR)])
        pltpu.sync_copy(m_hbm, mv)
        pltpu.sync_copy(sd_hbm.at[wid, 0], ix2[0])
        pltpu.async_copy(sd_hbm.at[wid, 1], ix2[1], six[1])
        pltpu.async_copy(als_hbm.at[ix2[0].at[0]], g1[0], sg1[0])
        pltpu.async_copy(ald_hbm.at[ix2[0].at[1]], g2[0], sg2[0])
        plsc.subcore_barrier()

        mvv = mv[0]
        is8 = lax.iota(i32, 16) == HEADS

        def quad(p4, _):
            for b4 in (0, 1, 2, 3):
                ci = 4 * p4 + b4
                rb = b4 % 2
                prb = 1 - rb
                ib = b4
                nib = (b4 + 1) % 4
                fib = (b4 + 2) % 4

                @pl.when(ci >= 1)
                def _():
                    pltpu.make_async_copy(exv[prb], ex_hbm.at[wid * chw + ci], sex[prb]).wait()
                    pltpu.make_async_copy(
                        exv[prb], den_sh.at[ix2[(b4 + 3) % 4].at[1]], ssc[prb]).wait()

                @pl.when(ci + 2 < chw)
                def _():
                    pltpu.async_copy(sd_hbm.at[wid, ci + 2], ix2[fib], six[fib])

                @pl.when(ci + 1 < chw)
                def _():
                    pltpu.make_async_copy(sd_hbm.at[wid, ci + 1], ix2[nib], six[nib]).wait()
                    pltpu.async_copy(als_hbm.at[ix2[nib].at[0]], g1[prb], sg1[prb])
                    pltpu.async_copy(ald_hbm.at[ix2[nib].at[1]], g2[prb], sg2[prb])

                pltpu.make_async_copy(als_hbm.at[ix2[ib].at[0]], g1[rb], sg1[rb]).wait()
                pltpu.make_async_copy(ald_hbm.at[ix2[ib].at[1]], g2[rb], sg2[rb]).wait()

                def row(i, _):
                    e = g1[rb][i] + g2[rb][i]
                    e = jnp.maximum(e, 0.2 * e)
                    ex = jnp.exp(e - mvv)
                    exv[rb][i] = jnp.where(is8, 1.0, ex)
                    return 0

                lax.fori_loop(0, K, row, 0)
                pltpu.async_copy(exv[rb], ex_hbm.at[wid * chw + ci], sex[rb])
                pltpu.async_copy(exv[rb], den_sh.at[ix2[ib].at[1]], ssc[rb], add=True)
            return 0

        lax.fori_loop(0, chw // 4, quad, 0)
        bl = (chw - 1) % 2
        pltpu.make_async_copy(exv[bl], ex_hbm.at[wid * chw + chw - 1], sex[bl]).wait()
        pltpu.make_async_copy(exv[bl], den_sh.at[ix2[(chw - 1) % 4].at[1]], ssc[bl]).wait()
        plsc.subcore_barrier()
        pltpu.sync_copy(den_sh.at[pl.ds(s * ZR, ZR)], zb)
        pltpu.sync_copy(zb, den_hbm.at[c, pl.ds(s * ZR, ZR)])

    return attn


def _make_agg_kernel(cht, scaled):
    @functools.partial(
        pl.kernel,
        out_type=jax.ShapeDtypeStruct((NC, NP_, DHALF), f32),
        mesh=_mesh,
        compiler_params=_sc_params,
        scratch_types=(
            pltpu.VMEM((2, K), i32),
            pltpu.VMEM((2, K), i32),
            pltpu.VMEM((2, K), i32),
            pltpu.VMEM((2, K), i32),
            pltpu.VMEM((K, DHALF), f32),
            pltpu.VMEM((K, DHALF), f32),
            pltpu.VMEM((K, 16), f32),
            pltpu.VMEM((K, 16), f32),
            pltpu.VMEM((ZR, DHALF), f32),
            pltpu.SemaphoreType.DMA,
            pltpu.SemaphoreType.DMA,
            pltpu.SemaphoreType.DMA,
            pltpu.SemaphoreType.DMA,
            pltpu.SemaphoreType.DMA,
            pltpu.SemaphoreType.DMA,
            pltpu.SemaphoreType.DMA,
            pltpu.SemaphoreType.DMA,
            pltpu.SemaphoreType.DMA,
            pltpu.SemaphoreType.DMA,
            pltpu.VMEM_SHARED((NP_, DHALF), f32),
        ),
    )
    def agg(tab_hbm, ex_hbm, sd_hbm, out_hbm,
            ixA, ixB, ixC, ixD, rowsa, rowsb, exa, exb2, zb,
            sixA, sixB, sixC, sixD, sga, sgb, sea, seb, ssa, ssb, acc_sh):
        c = lax.axis_index("c")
        s = lax.axis_index("s")
        ix2 = (ixA, ixB, ixC, ixD)
        six = (sixA, sixB, sixC, sixD)
        rows = (rowsa, rowsb)
        exv = (exa, exb2)
        sg = (sga, sgb)
        se = (sea, seb)
        ss = (ssa, ssb)
        zvec = jnp.zeros((16,), f32)

        def zrow(i, _):
            for j in range(DHALF // 16):
                zb[i, pl.ds(16 * j, 16)] = zvec
            return 0

        lax.fori_loop(0, ZR, zrow, 0)
        pltpu.sync_copy(zb, acc_sh.at[pl.ds(s * ZR, ZR)])
        pltpu.sync_copy(sd_hbm.at[s, 0], ix2[0])
        pltpu.async_copy(sd_hbm.at[s, 1], ix2[1], six[1])

        def gather(cc, ib, rb):
            def go():
                pltpu.async_copy(tab_hbm.at[cc].at[ix2[ib].at[0]], rows[rb], sg[rb])
            return go

        def gwait(cc, ib, rb):
            def go():
                pltpu.make_async_copy(
                    tab_hbm.at[cc].at[ix2[ib].at[0]], rows[rb], sg[rb]).wait()
            return go

        pl.when(c == 0)(gather(0, 0, 0))
        pl.when(c != 0)(gather(1, 0, 0))
        if scaled:
            pltpu.async_copy(ex_hbm.at[s * cht], exv[0], se[0])
        plsc.subcore_barrier()

        def quad(p4, _):
            for b4 in (0, 1, 2, 3):
                ci = 4 * p4 + b4
                rb = b4 % 2
                prb = 1 - rb
                ib = b4
                nib = (b4 + 1) % 4
                fib = (b4 + 2) % 4

                @pl.when(ci >= 1)
                def _():
                    pltpu.make_async_copy(
                        rows[prb], acc_sh.at[ix2[(b4 + 3) % 4].at[1]], ss[prb]).wait()

                @pl.when(ci + 2 < cht)
                def _():
                    pltpu.async_copy(sd_hbm.at[s, ci + 2], ix2[fib], six[fib])

                @pl.when(ci + 1 < cht)
                def _():
                    pltpu.make_async_copy(sd_hbm.at[s, ci + 1], ix2[nib], six[nib]).wait()
                    pl.when(c == 0)(gather(0, nib, prb))
                    pl.when(c != 0)(gather(1, nib, prb))
                    if scaled:
                        pltpu.async_copy(ex_hbm.at[s * cht + ci + 1], exv[prb], se[prb])

                pl.when(c == 0)(gwait(0, ib, rb))
                pl.when(c != 0)(gwait(1, ib, rb))
                if scaled:
                    pltpu.make_async_copy(ex_hbm.at[s * cht + ci], exv[rb], se[rb]).wait()

                    def scale(off):
                        def run():
                            def row(i, _):
                                exr = exv[rb][i]
                                for j in range(DHALF // 16):
                                    sl = pl.ds(16 * j, 16)
                                    rows[rb][i, sl] = rows[rb][i, sl] * exr[off + j]
                                return 0

                            lax.fori_loop(0, K, row, 0)
                        return run

                    pl.when(c == 0)(scale(0))
                    pl.when(c != 0)(scale(4))
                pltpu.async_copy(rows[rb], acc_sh.at[ix2[ib].at[1]], ss[rb], add=True)
            return 0

        lax.fori_loop(0, cht // 4, quad, 0)
        bl = (cht - 1) % 2
        pltpu.make_async_copy(rows[bl], acc_sh.at[ix2[(cht - 1) % 4].at[1]], ss[bl]).wait()
        plsc.subcore_barrier()
        pltpu.sync_copy(acc_sh.at[pl.ds(s * ZR, ZR)], zb)
        pltpu.sync_copy(zb, out_hbm.at[c, pl.ds(s * ZR, ZR)])

    return agg


def kernel(x, edge_index, W_gat, a_src, a_dst, W_gcn):
    E = edge_index.shape[1]
    E2 = E + N
    EP = -(-E2 // (4 * NW * K)) * (4 * NW * K)
    PAD = EP - E2
    CHW = EP // (NW * K)
    CHT = EP // (NS * K)

    xp = jnp.pad(x, ((0, NP_ - N), (0, 0)))
    eye8 = jnp.eye(HEADS, dtype=f32)
    a1 = jnp.concatenate(
        [(eye8[:, None, :] * a_src[:, :, None]).reshape(DH, HEADS),
         jnp.zeros((DH, HEADS), f32)], axis=1)
    a2 = jnp.concatenate(
        [(eye8[:, None, :] * a_dst[:, :, None]).reshape(DH, HEADS),
         jnp.zeros((DH, HEADS), f32)], axis=1)
    r1 = jnp.concatenate(
        [jnp.kron(eye8, jnp.ones((1, HID), f32)), jnp.zeros((8, DH), f32)],
        axis=0)
    r2 = jnp.zeros((16, DH), f32).at[HEADS].set(1.0)

    loop = jnp.arange(N, dtype=i32)
    padv = jnp.full((PAD,), N, dtype=i32)
    srcp = jnp.concatenate([edge_index[0].astype(i32), loop, padv])
    dstp = jnp.concatenate([edge_index[1].astype(i32), loop, padv])
    sd_attn = jnp.concatenate(
        [srcp.reshape(NW, CHW, 1, K), dstp.reshape(NW, CHW, 1, K)], axis=2)
    sd_agg = jnp.concatenate(
        [srcp.reshape(NS, CHT, 1, K), dstp.reshape(NS, CHT, 1, K)], axis=2)

    hh, als, ald, m16 = pl.pallas_call(
        _k0_body,
        out_shape=(
            jax.ShapeDtypeStruct((NC, NP_, DHALF), f32),
            jax.ShapeDtypeStruct((NP_, 16), f32),
            jax.ShapeDtypeStruct((NP_, 16), f32),
            jax.ShapeDtypeStruct((1, 16), f32),
        ),
    )(xp, W_gat, a1, a2)

    ex3, den2 = _make_attn_kernel(CHW)(als, ald, m16, sd_attn)
    ex_agg = ex3.reshape(NS * CHT, K, 16)
    acc_gat = _make_agg_kernel(CHT, scaled=True)(hh, ex_agg, sd_agg)

    h2h = pl.pallas_call(
        _kd_body,
        out_shape=jax.ShapeDtypeStruct((NC, NP_, DHALF), f32),
    )(acc_gat, den2, W_gcn, r1, r2)

    acc_gcn = _make_agg_kernel(CHT, scaled=False)(h2h, ex_agg, sd_agg)

    y = pl.pallas_call(
        _ke_body,
        out_shape=jax.ShapeDtypeStruct((NP_, D_OUT), f32),
    )(acc_gcn, den2, r2)

    return y[:N]


# speedup vs baseline: 1.0380x; 1.0380x over previous
import functools

import jax
import jax.numpy as jnp
import numpy as np
from jax import lax
from jax.experimental import pallas as pl
from jax.experimental.pallas import tpu as pltpu
from jax.experimental.pallas import tpu_sc as plsc

N = 10000
D_IN = 128
HEADS = 8
HID = 16
DH = HEADS * HID
DHALF = DH // 2
D_OUT = 128

NP_ = 10112
NC = 2
NS = 16
NW = NC * NS
K = 128
ZR = NP_ // NS

_mesh = plsc.VectorSubcoreMesh(
    core_axis_name="c", subcore_axis_name="s", num_cores=NC, num_subcores=NS
)
_sc_params = pltpu.CompilerParams(use_tc_tiling_on_sc=False)

f32 = jnp.float32
i32 = jnp.int32


def _k0_body(x_ref, wg_ref, a1_ref, a2_ref, hh_ref, als_ref, ald_ref, m_ref):
    h = jnp.dot(x_ref[...], wg_ref[...], preferred_element_type=f32)
    hh_ref[0] = h[:, :DHALF]
    hh_ref[1] = h[:, DHALF:]
    als = jnp.dot(h, a1_ref[...], preferred_element_type=f32)
    ald = jnp.dot(h, a2_ref[...], preferred_element_type=f32)
    als_ref[...] = als
    ald_ref[...] = ald
    s = jnp.max(als, axis=0, keepdims=True) + jnp.max(ald, axis=0, keepdims=True)
    mlr = jnp.maximum(s, 0.2 * s)
    col = lax.broadcasted_iota(i32, (1, 16), 1)
    m_ref[...] = jnp.where(col < HEADS, mlr, 1e9)


def _kd_body(acc_ref, den_ref, wgcn_ref, r1_ref, r2_ref, out_ref):
    acc = jnp.concatenate([acc_ref[0], acc_ref[1]], axis=1)
    den_all = den_ref[0] + den_ref[1]
    den128 = jnp.dot(den_all, r1_ref[...], preferred_element_type=f32)
    deg128 = jnp.dot(den_all, r2_ref[...], preferred_element_type=f32)
    ygat = acc / (den128 + 1e-16)
    dinv = jnp.where(deg128 > 0, lax.rsqrt(deg128), 0.0)
    h2p = dinv * jnp.dot(ygat, wgcn_ref[...], preferred_element_type=f32)
    out_ref[0] = h2p[:, :DHALF]
    out_ref[1] = h2p[:, DHALF:]


def _ke_body(acc_ref, den_ref, r2_ref, out_ref):
    den_all = den_ref[0] + den_ref[1]
    deg128 = jnp.dot(den_all, r2_ref[...], preferred_element_type=f32)
    dinv = jnp.where(deg128 > 0, lax.rsqrt(deg128), 0.0)
    out_ref[...] = dinv * jnp.concatenate([acc_ref[0], acc_ref[1]], axis=1)


def _make_attn_kernel(chw):
    @functools.partial(
        pl.kernel,
        out_type=(
            jax.ShapeDtypeStruct((NW * chw, K, 16), f32),
            jax.ShapeDtypeStruct((NC, NP_, 16), f32),
        ),
        mesh=_mesh,
        compiler_params=_sc_params,
        scratch_types=(
            pltpu.VMEM((2, K), i32),
            pltpu.VMEM((2, K), i32),
            pltpu.VMEM((2, K), i32),
            pltpu.VMEM((2, K), i32),
            pltpu.SemaphoreType.DMA,
            pltpu.SemaphoreType.DMA,
            pltpu.SemaphoreType.DMA,
            pltpu.SemaphoreType.DMA,
            pltpu.VMEM((K, 16), f32),
            pltpu.VMEM((K, 16), f32),
            pltpu.VMEM((K, 16), f32),
            pltpu.VMEM((K, 16), f32),
            pltpu.VMEM((K, 16), f32),
            pltpu.VMEM((K, 16), f32),
            pltpu.VMEM((K, 16), f32),
            pltpu.VMEM((K, 16), f32),
            pltpu.VMEM((K, 16), f32),
            pltpu.VMEM((K, 16), f32),
            pltpu.VMEM((K, 16), f32),
            pltpu.VMEM((K, 16), f32),
            pltpu.VMEM((1, 16), f32),
            pltpu.VMEM((ZR, 16), f32),
            pltpu.SemaphoreType.DMA,
            pltpu.SemaphoreType.DMA,
            pltpu.SemaphoreType.DMA,
            pltpu.SemaphoreType.DMA,
            pltpu.SemaphoreType.DMA,
            pltpu.SemaphoreType.DMA,
            pltpu.SemaphoreType.DMA,
            pltpu.SemaphoreType.DMA,
            pltpu.SemaphoreType.DMA,
            pltpu.SemaphoreType.DMA,
            pltpu.SemaphoreType.DMA,
            pltpu.SemaphoreType.DMA,
            pltpu.SemaphoreType.DMA,
            pltpu.SemaphoreType.DMA,
            pltpu.SemaphoreType.DMA,
            pltpu.SemaphoreType.DMA,
            pltpu.VMEM_SHARED((NP_, 16), f32),
        ),
    )
    def attn(als_hbm, ald_hbm, m_hbm, sd_hbm, ex_hbm, den_hbm,
             ix2a, ix2b, ix2c, ix2d, sixa, sixb, sixc, sixd,
             g1a, g1b, g1c, g1d, g2a, g2b, g2c, g2d,
             exa, exb2, exc, exd, mv, zb,
             sg1a, sg1b, sg1c, sg1d, sg2a, sg2b, sg2c, sg2d,
             sexa, sexb, sexc, sexd, ssca, sscb, sscc, sscd, den_sh):
        c = lax.axis_index("c")
        s = lax.axis_index("s")
        wid = s * NC + c
        ix2 = (ix2a, ix2b, ix2c, ix2d)
        six = (sixa, sixb, sixc, sixd)
        g1 = (g1a, g1b, g1c, g1d)
        g2 = (g2a, g2b, g2c, g2d)
        exv = (exa, exb2, exc, exd)
        sg1 = (sg1a, sg1b, sg1c, sg1d)
        sg2 = (sg2a, sg2b, sg2c, sg2d)
        sex = (sexa, sexb, sexc, sexd)
        ssc = (ssca, sscb, sscc, sscd)
        zvec = jnp.zeros((16,), f32)

        def zrow(i, _):
            zb[i] = zvec
            return 0

        lax.fori_loop(0, ZR, zrow, 0)
        pltpu.sync_copy(zb, den_sh.at[pl.ds(s * ZR, ZR)])
        pltpu.sync_copy(m_hbm, mv)
        pltpu.sync_copy(sd_hbm.at[wid, 0], ix2[0])
        pltpu.async_copy(sd_hbm.at[wid, 1], ix2[1], six[1])
        pltpu.async_copy(als_hbm.at[ix2[0].at[0]], g1[0], sg1[0])
        pltpu.async_copy(ald_hbm.at[ix2[0].at[1]], g2[0], sg2[0])
        plsc.subcore_barrier()

        mvv = mv[0]
        is8 = lax.iota(i32, 16) == HEADS

        def quad(p4, _):
            for b4 in (0, 1, 2, 3):
                ci = 4 * p4 + b4
                rb = b4
                nib = (b4 + 1) % 4
                fib = (b4 + 2) % 4

                @pl.when(ci >= 2)
                def _():
                    pltpu.make_async_copy(exv[fib], ex_hbm.at[wid * chw + ci], sex[fib]).wait()
                    pltpu.make_async_copy(
                        exv[fib], den_sh.at[ix2[fib].at[1]], ssc[fib]).wait()

                @pl.when(ci + 2 < chw)
                def _():
                    pltpu.async_copy(sd_hbm.at[wid, ci + 2], ix2[fib], six[fib])

                @pl.when(ci + 1 < chw)
                def _():
                    pltpu.make_async_copy(sd_hbm.at[wid, ci + 1], ix2[nib], six[nib]).wait()
                    pltpu.async_copy(als_hbm.at[ix2[nib].at[0]], g1[nib], sg1[nib])
                    pltpu.async_copy(ald_hbm.at[ix2[nib].at[1]], g2[nib], sg2[nib])

                pltpu.make_async_copy(als_hbm.at[ix2[rb].at[0]], g1[rb], sg1[rb]).wait()
                pltpu.make_async_copy(ald_hbm.at[ix2[rb].at[1]], g2[rb], sg2[rb]).wait()

                def row(i, _):
                    e = g1[rb][i] + g2[rb][i]
                    e = jnp.maximum(e, 0.2 * e)
                    ex = jnp.exp(e - mvv)
                    exv[rb][i] = jnp.where(is8, 1.0, ex)
                    return 0

                lax.fori_loop(0, K, row, 0)
                pltpu.async_copy(exv[rb], ex_hbm.at[wid * chw + ci], sex[rb])
                pltpu.async_copy(exv[rb], den_sh.at[ix2[rb].at[1]], ssc[rb], add=True)
            return 0

        lax.fori_loop(0, chw // 4, quad, 0)
        for tl in (2, 3):
            ci = chw - 4 + tl
            pltpu.make_async_copy(exv[tl], ex_hbm.at[wid * chw + ci], sex[tl]).wait()
            pltpu.make_async_copy(exv[tl], den_sh.at[ix2[tl].at[1]], ssc[tl]).wait()
        plsc.subcore_barrier()
        pltpu.sync_copy(den_sh.at[pl.ds(s * ZR, ZR)], zb)
        pltpu.sync_copy(zb, den_hbm.at[c, pl.ds(s * ZR, ZR)])

    return attn


def _make_agg_kernel(cht, scaled):
    @functools.partial(
        pl.kernel,
        out_type=jax.ShapeDtypeStruct((NC, NP_, DHALF), f32),
        mesh=_mesh,
        compiler_params=_sc_params,
        scratch_types=(
            pltpu.VMEM((2, K), i32),
            pltpu.VMEM((2, K), i32),
            pltpu.VMEM((2, K), i32),
            pltpu.VMEM((2, K), i32),
            pltpu.VMEM((K, DHALF), f32),
            pltpu.VMEM((K, DHALF), f32),
            pltpu.VMEM((K, DHALF), f32),
            pltpu.VMEM((K, DHALF), f32),
            pltpu.VMEM((K, 16), f32),
            pltpu.VMEM((K, 16), f32),
            pltpu.VMEM((K, 16), f32),
            pltpu.VMEM((K, 16), f32),
            pltpu.VMEM((ZR, DHALF), f32),
            pltpu.SemaphoreType.DMA,
            pltpu.SemaphoreType.DMA,
            pltpu.SemaphoreType.DMA,
            pltpu.SemaphoreType.DMA,
            pltpu.SemaphoreType.DMA,
            pltpu.SemaphoreType.DMA,
            pltpu.SemaphoreType.DMA,
            pltpu.SemaphoreType.DMA,
            pltpu.SemaphoreType.DMA,
            pltpu.SemaphoreType.DMA,
            pltpu.SemaphoreType.DMA,
            pltpu.SemaphoreType.DMA,
            pltpu.SemaphoreType.DMA,
            pltpu.SemaphoreType.DMA,
            pltpu.SemaphoreType.DMA,
            pltpu.SemaphoreType.DMA,
            pltpu.VMEM_SHARED((NP_, DHALF), f32),
        ),
    )
    def agg(tab_hbm, ex_hbm, sd_hbm, out_hbm,
            ixA, ixB, ixC, ixD, rowsa, rowsb, rowsc, rowsd,
            exa, exb2, exc, exd, zb,
            sixA, sixB, sixC, sixD, sga, sgb, sgc, sgd,
            sea, seb, sec, sed, ssa, ssb, ssc2, ssd, acc_sh):
        c = lax.axis_index("c")
        s = lax.axis_index("s")
        ix2 = (ixA, ixB, ixC, ixD)
        six = (sixA, sixB, sixC, sixD)
        rows = (rowsa, rowsb, rowsc, rowsd)
        exv = (exa, exb2, exc, exd)
        sg = (sga, sgb, sgc, sgd)
        se = (sea, seb, sec, sed)
        ss = (ssa, ssb, ssc2, ssd)
        zvec = jnp.zeros((16,), f32)

        def zrow(i, _):
            for j in range(DHALF // 16):
                zb[i, pl.ds(16 * j, 16)] = zvec
            return 0

        lax.fori_loop(0, ZR, zrow, 0)
        pltpu.sync_copy(zb, acc_sh.at[pl.ds(s * ZR, ZR)])
        pltpu.sync_copy(sd_hbm.at[s, 0], ix2[0])
        pltpu.async_copy(sd_hbm.at[s, 1], ix2[1], six[1])

        def gather(cc, ib, rb):
            def go():
                pltpu.async_copy(tab_hbm.at[cc].at[ix2[ib].at[0]], rows[rb], sg[rb])
            return go

        def gwait(cc, ib, rb):
            def go():
                pltpu.make_async_copy(
                    tab_hbm.at[cc].at[ix2[ib].at[0]], rows[rb], sg[rb]).wait()
            return go

        pl.when(c == 0)(gather(0, 0, 0))
        pl.when(c != 0)(gather(1, 0, 0))
        if scaled:
            pltpu.async_copy(ex_hbm.at[s * cht], exv[0], se[0])
        plsc.subcore_barrier()

        def quad(p4, _):
            for b4 in (0, 1, 2, 3):
                ci = 4 * p4 + b4
                rb = b4
                nib = (b4 + 1) % 4
                fib = (b4 + 2) % 4

                @pl.when(ci >= 2)
                def _():
                    pltpu.make_async_copy(
                        rows[fib], acc_sh.at[ix2[fib].at[1]], ss[fib]).wait()

                @pl.when(ci + 2 < cht)
                def _():
                    pltpu.async_copy(sd_hbm.at[s, ci + 2], ix2[fib], six[fib])

                @pl.when(ci + 1 < cht)
                def _():
                    pltpu.make_async_copy(sd_hbm.at[s, ci + 1], ix2[nib], six[nib]).wait()
                    pl.when(c == 0)(gather(0, nib, nib))
                    pl.when(c != 0)(gather(1, nib, nib))
                    if scaled:
                        pltpu.async_copy(ex_hbm.at[s * cht + ci + 1], exv[nib], se[nib])

                pl.when(c == 0)(gwait(0, rb, rb))
                pl.when(c != 0)(gwait(1, rb, rb))
                if scaled:
                    pltpu.make_async_copy(ex_hbm.at[s * cht + ci], exv[rb], se[rb]).wait()

                    def scale(off):
                        def run():
                            def row(i, _):
                                exr = exv[rb][i]
                                for j in range(DHALF // 16):
                                    sl = pl.ds(16 * j, 16)
                                    rows[rb][i, sl] = rows[rb][i, sl] * exr[off + j]
                                return 0

                            lax.fori_loop(0, K, row, 0)
                        return run

                    pl.when(c == 0)(scale(0))
                    pl.when(c != 0)(scale(4))
                pltpu.async_copy(rows[rb], acc_sh.at[ix2[rb].at[1]], ss[rb], add=True)
            return 0

        lax.fori_loop(0, cht // 4, quad, 0)
        for tl in (2, 3):
            ci = cht - 4 + tl
            pltpu.make_async_copy(rows[tl], acc_sh.at[ix2[tl].at[1]], ss[tl]).wait()
        plsc.subcore_barrier()
        pltpu.sync_copy(acc_sh.at[pl.ds(s * ZR, ZR)], zb)
        pltpu.sync_copy(zb, out_hbm.at[c, pl.ds(s * ZR, ZR)])

    return agg


def kernel(x, edge_index, W_gat, a_src, a_dst, W_gcn):
    E = edge_index.shape[1]
    E2 = E + N
    EP = -(-E2 // (4 * NW * K)) * (4 * NW * K)
    PAD = EP - E2
    CHW = EP // (NW * K)
    CHT = EP // (NS * K)

    xp = jnp.pad(x, ((0, NP_ - N), (0, 0)))
    eye8 = jnp.eye(HEADS, dtype=f32)
    a1 = jnp.concatenate(
        [(eye8[:, None, :] * a_src[:, :, None]).reshape(DH, HEADS),
         jnp.zeros((DH, HEADS), f32)], axis=1)
    a2 = jnp.concatenate(
        [(eye8[:, None, :] * a_dst[:, :, None]).reshape(DH, HEADS),
         jnp.zeros((DH, HEADS), f32)], axis=1)
    r1 = jnp.concatenate(
        [jnp.kron(eye8, jnp.ones((1, HID), f32)), jnp.zeros((8, DH), f32)],
        axis=0)
    r2 = jnp.zeros((16, DH), f32).at[HEADS].set(1.0)

    loop = jnp.arange(N, dtype=i32)
    padv_s = jnp.full((PAD,), N, dtype=i32)
    padv_d = N + (jnp.arange(PAD, dtype=i32) % (NP_ - N))
    srcp = jnp.concatenate([edge_index[0].astype(i32), loop, padv_s])
    dstp = jnp.concatenate([edge_index[1].astype(i32), loop, padv_d])
    sd_attn = jnp.concatenate(
        [srcp.reshape(NW, CHW, 1, K), dstp.reshape(NW, CHW, 1, K)], axis=2)
    sd_agg = jnp.concatenate(
        [srcp.reshape(NS, CHT, 1, K), dstp.reshape(NS, CHT, 1, K)], axis=2)

    hh, als, ald, m16 = pl.pallas_call(
        _k0_body,
        out_shape=(
            jax.ShapeDtypeStruct((NC, NP_, DHALF), f32),
            jax.ShapeDtypeStruct((NP_, 16), f32),
            jax.ShapeDtypeStruct((NP_, 16), f32),
            jax.ShapeDtypeStruct((1, 16), f32),
        ),
    )(xp, W_gat, a1, a2)

    ex3, den2 = _make_attn_kernel(CHW)(als, ald, m16, sd_attn)
    ex_agg = ex3.reshape(NS * CHT, K, 16)
    acc_gat = _make_agg_kernel(CHT, scaled=True)(hh, ex_agg, sd_agg)

    h2h = pl.pallas_call(
        _kd_body,
        out_shape=jax.ShapeDtypeStruct((NC, NP_, DHALF), f32),
    )(acc_gat, den2, W_gcn, r1, r2)

    acc_gcn = _make_agg_kernel(CHT, scaled=False)(h2h, ex_agg, sd_agg)

    y = pl.pallas_call(
        _ke_body,
        out_shape=jax.ShapeDtypeStruct((NP_, D_OUT), f32),
    )(acc_gcn, den2, r2)

    return y[:N]


# speedup vs baseline: 1.4305x; 1.3781x over previous
import functools

import jax
import jax.numpy as jnp
import numpy as np
from jax import lax
from jax.experimental import pallas as pl
from jax.experimental.pallas import tpu as pltpu
from jax.experimental.pallas import tpu_sc as plsc

N = 10000
D_IN = 128
HEADS = 8
HID = 16
DH = HEADS * HID
DHALF = DH // 2
D_OUT = 128

NP_ = 10112
NC = 2
NS = 16
NW = NC * NS
K = 128
ZR = NP_ // NS

_mesh = plsc.VectorSubcoreMesh(
    core_axis_name="c", subcore_axis_name="s", num_cores=NC, num_subcores=NS
)
_sc_params = pltpu.CompilerParams(use_tc_tiling_on_sc=False)

f32 = jnp.float32
i32 = jnp.int32


def _k0_body(x_ref, wg_ref, a1_ref, a2_ref, hh_ref, als_ref, ald_ref, m_ref):
    h = jnp.dot(x_ref[...], wg_ref[...], preferred_element_type=f32)
    hh_ref[0] = h[:, :DHALF]
    hh_ref[1] = h[:, DHALF:]
    als = jnp.dot(h, a1_ref[...], preferred_element_type=f32)
    ald = jnp.dot(h, a2_ref[...], preferred_element_type=f32)
    als_ref[...] = als
    ald_ref[...] = ald
    s = jnp.max(als, axis=0, keepdims=True) + jnp.max(ald, axis=0, keepdims=True)
    mlr = jnp.maximum(s, 0.2 * s)
    col = lax.broadcasted_iota(i32, (1, 16), 1)
    m_ref[...] = jnp.where(col < HEADS, mlr, 1e9)


def _kd_body(acc_ref, den_ref, wgcn_ref, r1_ref, r2_ref, out_ref):
    acc = jnp.concatenate([acc_ref[0], acc_ref[1]], axis=1)
    den_all = den_ref[0] + den_ref[1]
    den128 = jnp.dot(den_all, r1_ref[...], preferred_element_type=f32)
    deg128 = jnp.dot(den_all, r2_ref[...], preferred_element_type=f32)
    ygat = acc / (den128 + 1e-16)
    dinv = jnp.where(deg128 > 0, lax.rsqrt(deg128), 0.0)
    h2p = dinv * jnp.dot(ygat, wgcn_ref[...], preferred_element_type=f32)
    out_ref[0] = h2p[:, :DHALF]
    out_ref[1] = h2p[:, DHALF:]


def _ke_body(acc_ref, den_ref, r2_ref, out_ref):
    den_all = den_ref[0] + den_ref[1]
    deg128 = jnp.dot(den_all, r2_ref[...], preferred_element_type=f32)
    dinv = jnp.where(deg128 > 0, lax.rsqrt(deg128), 0.0)
    out_ref[...] = dinv * jnp.concatenate([acc_ref[0], acc_ref[1]], axis=1)


def _make_attn_kernel(chw):
    @functools.partial(
        pl.kernel,
        out_type=(
            jax.ShapeDtypeStruct((NW * chw, K, 16), f32),
            jax.ShapeDtypeStruct((NC, NP_, 16), f32),
        ),
        mesh=_mesh,
        compiler_params=_sc_params,
        scratch_types=(
            pltpu.VMEM((2, K), i32),
            pltpu.VMEM((2, K), i32),
            pltpu.VMEM((K, 16), f32),
            pltpu.VMEM((K, 16), f32),
            pltpu.VMEM((K, 16), f32),
            pltpu.VMEM((K, 16), f32),
            pltpu.VMEM((K, 16), f32),
            pltpu.VMEM((K, 16), f32),
            pltpu.VMEM((1, 16), f32),
            pltpu.VMEM((ZR, 16), f32),
            pltpu.SemaphoreType.DMA,
            pltpu.SemaphoreType.DMA,
            pltpu.SemaphoreType.DMA,
            pltpu.SemaphoreType.DMA,
            pltpu.SemaphoreType.DMA,
            pltpu.SemaphoreType.DMA,
            pltpu.SemaphoreType.DMA,
            pltpu.SemaphoreType.DMA,
            pltpu.VMEM_SHARED((NP_, 16), f32),
        ),
    )
    def attn(als_hbm, ald_hbm, m_hbm, sd_hbm, ex_hbm, den_hbm,
             ix2a, ix2b, g1a, g1b, g2a, g2b, exa, exb2, mv, zb,
             sg1a, sg1b, sg2a, sg2b, sexa, sexb, ssca, sscb, den_sh):
        c = lax.axis_index("c")
        s = lax.axis_index("s")
        wid = s * NC + c
        ix2 = (ix2a, ix2b)
        g1 = (g1a, g1b)
        g2 = (g2a, g2b)
        exv = (exa, exb2)
        sg1 = (sg1a, sg1b)
        sg2 = (sg2a, sg2b)
        sex = (sexa, sexb)
        ssc = (ssca, sscb)
        zvec = jnp.zeros((16,), f32)

        def zrow(i, _):
            zb[i] = zvec
            return 0

        lax.fori_loop(0, ZR, zrow, 0)
        pltpu.sync_copy(zb, den_sh.at[pl.ds(s * ZR, ZR)])
        pltpu.sync_copy(m_hbm, mv)
        pltpu.sync_copy(sd_hbm.at[wid, 0], ix2[0])
        pltpu.async_copy(als_hbm.at[ix2[0].at[0]], g1[0], sg1[0])
        pltpu.async_copy(ald_hbm.at[ix2[0].at[1]], g2[0], sg2[0])
        plsc.subcore_barrier()

        mvv = mv[0]
        is8 = lax.iota(i32, 16) == HEADS

        def pair(p, _):
            for b in (0, 1):
                ci = 2 * p + b
                nb = 1 - b

                @pl.when(ci >= 1)
                def _():
                    pltpu.make_async_copy(exv[nb], ex_hbm.at[wid * chw + ci], sex[nb]).wait()
                    pltpu.make_async_copy(exv[nb], den_sh.at[ix2[nb].at[1]], ssc[nb]).wait()

                @pl.when(ci + 1 < chw)
                def _():
                    pltpu.sync_copy(sd_hbm.at[wid, ci + 1], ix2[nb])
                    pltpu.async_copy(als_hbm.at[ix2[nb].at[0]], g1[nb], sg1[nb])
                    pltpu.async_copy(ald_hbm.at[ix2[nb].at[1]], g2[nb], sg2[nb])

                pltpu.make_async_copy(als_hbm.at[ix2[b].at[0]], g1[b], sg1[b]).wait()
                pltpu.make_async_copy(ald_hbm.at[ix2[b].at[1]], g2[b], sg2[b]).wait()

                def row(i, _):
                    e = g1[b][i] + g2[b][i]
                    e = jnp.maximum(e, 0.2 * e)
                    ex = jnp.exp(e - mvv)
                    exv[b][i] = jnp.where(is8, 1.0, ex)
                    return 0

                lax.fori_loop(0, K, row, 0)
                pltpu.async_copy(exv[b], ex_hbm.at[wid * chw + ci], sex[b])
                pltpu.async_copy(exv[b], den_sh.at[ix2[b].at[1]], ssc[b], add=True)
            return 0

        lax.fori_loop(0, chw // 2, pair, 0)
        bl = (chw - 1) % 2
        pltpu.make_async_copy(exv[bl], ex_hbm.at[wid * chw + chw - 1], sex[bl]).wait()
        pltpu.make_async_copy(exv[bl], den_sh.at[ix2[bl].at[1]], ssc[bl]).wait()
        plsc.subcore_barrier()
        pltpu.sync_copy(den_sh.at[pl.ds(s * ZR, ZR)], zb)
        pltpu.sync_copy(zb, den_hbm.at[c, pl.ds(s * ZR, ZR)])

    return attn


def _make_agg_kernel(cht, scaled):
    @functools.partial(
        pl.kernel,
        out_type=jax.ShapeDtypeStruct((NC, NP_, DHALF), f32),
        mesh=_mesh,
        compiler_params=_sc_params,
        scratch_types=(
            pltpu.VMEM((2, K), i32),
            pltpu.VMEM((2, K), i32),
            pltpu.VMEM((2, K), i32),
            pltpu.VMEM((2, K), i32),
            pltpu.VMEM((K, DHALF), f32),
            pltpu.VMEM((K, DHALF), f32),
            pltpu.VMEM((K, 16), f32),
            pltpu.VMEM((K, 16), f32),
            pltpu.VMEM((ZR, DHALF), f32),
            pltpu.SemaphoreType.DMA,
            pltpu.SemaphoreType.DMA,
            pltpu.SemaphoreType.DMA,
            pltpu.SemaphoreType.DMA,
            pltpu.SemaphoreType.DMA,
            pltpu.SemaphoreType.DMA,
            pltpu.SemaphoreType.DMA,
            pltpu.SemaphoreType.DMA,
            pltpu.SemaphoreType.DMA,
            pltpu.SemaphoreType.DMA,
            pltpu.VMEM_SHARED((NP_, DHALF), f32),
        ),
    )
    def agg(tab_hbm, ex_hbm, sd_hbm, out_hbm,
            ixA, ixB, ixC, ixD, rowsa, rowsb, exa, exb2, zb,
            sixA, sixB, sixC, sixD, sga, sgb, sea, seb, ssa, ssb, acc_sh):
        c = lax.axis_index("c")
        s = lax.axis_index("s")
        ix2 = (ixA, ixB, ixC, ixD)
        six = (sixA, sixB, sixC, sixD)
        rows = (rowsa, rowsb)
        exv = (exa, exb2)
        sg = (sga, sgb)
        se = (sea, seb)
        ss = (ssa, ssb)
        zvec = jnp.zeros((16,), f32)

        def zrow(i, _):
            for j in range(DHALF // 16):
                zb[i, pl.ds(16 * j, 16)] = zvec
            return 0

        lax.fori_loop(0, ZR, zrow, 0)
        pltpu.sync_copy(zb, acc_sh.at[pl.ds(s * ZR, ZR)])
        pltpu.sync_copy(sd_hbm.at[s, 0], ix2[0])
        pltpu.async_copy(sd_hbm.at[s, 1], ix2[1], six[1])

        def gather(cc, ib, rb):
            def go():
                pltpu.async_copy(tab_hbm.at[cc].at[ix2[ib].at[0]], rows[rb], sg[rb])
            return go

        def gwait(cc, ib, rb):
            def go():
                pltpu.make_async_copy(
                    tab_hbm.at[cc].at[ix2[ib].at[0]], rows[rb], sg[rb]).wait()
            return go

        pl.when(c == 0)(gather(0, 0, 0))
        pl.when(c != 0)(gather(1, 0, 0))
        if scaled:
            pltpu.async_copy(ex_hbm.at[s * cht], exv[0], se[0])
        plsc.subcore_barrier()

        def quad(p4, _):
            for b4 in (0, 1, 2, 3):
                ci = 4 * p4 + b4
                rb = b4 % 2
                prb = 1 - rb
                ib = b4
                nib = (b4 + 1) % 4
                fib = (b4 + 2) % 4

                @pl.when(ci >= 1)
                def _():
                    pltpu.make_async_copy(
                        rows[prb], acc_sh.at[ix2[(b4 + 3) % 4].at[1]], ss[prb]).wait()

                @pl.when(ci + 2 < cht)
                def _():
                    pltpu.async_copy(sd_hbm.at[s, ci + 2], ix2[fib], six[fib])

                @pl.when(ci + 1 < cht)
                def _():
                    pltpu.make_async_copy(sd_hbm.at[s, ci + 1], ix2[nib], six[nib]).wait()
                    pl.when(c == 0)(gather(0, nib, prb))
                    pl.when(c != 0)(gather(1, nib, prb))
                    if scaled:
                        pltpu.async_copy(ex_hbm.at[s * cht + ci + 1], exv[prb], se[prb])

                pl.when(c == 0)(gwait(0, ib, rb))
                pl.when(c != 0)(gwait(1, ib, rb))
                if scaled:
                    pltpu.make_async_copy(ex_hbm.at[s * cht + ci], exv[rb], se[rb]).wait()

                    def scale(off):
                        def run():
                            def row(i, _):
                                exr = exv[rb][i]
                                for j in range(DHALF // 16):
                                    sl = pl.ds(16 * j, 16)
                                    rows[rb][i, sl] = rows[rb][i, sl] * exr[off + j]
                                return 0

                            lax.fori_loop(0, K, row, 0)
                        return run

                    pl.when(c == 0)(scale(0))
                    pl.when(c != 0)(scale(4))
                pltpu.async_copy(rows[rb], acc_sh.at[ix2[ib].at[1]], ss[rb], add=True)
            return 0

        lax.fori_loop(0, cht // 4, quad, 0)
        bl = (cht - 1) % 2
        pltpu.make_async_copy(rows[bl], acc_sh.at[ix2[(cht - 1) % 4].at[1]], ss[bl]).wait()
        plsc.subcore_barrier()
        pltpu.sync_copy(acc_sh.at[pl.ds(s * ZR, ZR)], zb)
        pltpu.sync_copy(zb, out_hbm.at[c, pl.ds(s * ZR, ZR)])

    return agg


def kernel(x, edge_index, W_gat, a_src, a_dst, W_gcn):
    E = edge_index.shape[1]
    E2 = E + N
    EP = -(-E2 // (2 * NW * K)) * (2 * NW * K)
    PAD = EP - E2
    CHW = EP // (NW * K)
    CHT = EP // (NS * K)

    xp = jnp.pad(x, ((0, NP_ - N), (0, 0)))
    eye8 = jnp.eye(HEADS, dtype=f32)
    a1 = jnp.concatenate(
        [(eye8[:, None, :] * a_src[:, :, None]).reshape(DH, HEADS),
         jnp.zeros((DH, HEADS), f32)], axis=1)
    a2 = jnp.concatenate(
        [(eye8[:, None, :] * a_dst[:, :, None]).reshape(DH, HEADS),
         jnp.zeros((DH, HEADS), f32)], axis=1)
    r1 = jnp.concatenate(
        [jnp.kron(eye8, jnp.ones((1, HID), f32)), jnp.zeros((8, DH), f32)],
        axis=0)
    r2 = jnp.zeros((16, DH), f32).at[HEADS].set(1.0)

    loop = jnp.arange(N, dtype=i32)
    padv_s = jnp.full((PAD,), N, dtype=i32)
    padv_d = N + (jnp.arange(PAD, dtype=i32) % (NP_ - N))
    srcp = jnp.concatenate([edge_index[0].astype(i32), loop, padv_s])
    dstp = jnp.concatenate([edge_index[1].astype(i32), loop, padv_d])
    sd_attn = jnp.concatenate(
        [srcp.reshape(NW, CHW, 1, K), dstp.reshape(NW, CHW, 1, K)], axis=2)
    sd_agg = jnp.concatenate(
        [srcp.reshape(NS, CHT, 1, K), dstp.reshape(NS, CHT, 1, K)], axis=2)

    hh, als, ald, m16 = pl.pallas_call(
        _k0_body,
        out_shape=(
            jax.ShapeDtypeStruct((NC, NP_, DHALF), f32),
            jax.ShapeDtypeStruct((NP_, 16), f32),
            jax.ShapeDtypeStruct((NP_, 16), f32),
            jax.ShapeDtypeStruct((1, 16), f32),
        ),
    )(xp, W_gat, a1, a2)

    ex3, den2 = _make_attn_kernel(CHW)(als, ald, m16, sd_attn)
    ex_agg = ex3.reshape(NS * CHT, K, 16)
    acc_gat = _make_agg_kernel(CHT, scaled=True)(hh, ex_agg, sd_agg)

    h2h = pl.pallas_call(
        _kd_body,
        out_shape=jax.ShapeDtypeStruct((NC, NP_, DHALF), f32),
    )(acc_gat, den2, W_gcn, r1, r2)

    acc_gcn = _make_agg_kernel(CHT, scaled=False)(h2h, ex_agg, sd_agg)

    y = pl.pallas_call(
        _ke_body,
        out_shape=jax.ShapeDtypeStruct((NP_, D_OUT), f32),
    )(acc_gcn, den2, r2)

    return y[:N]


# speedup vs baseline: 1.4682x; 1.0264x over previous
import functools

import jax
import jax.numpy as jnp
import numpy as np
from jax import lax
from jax.experimental import pallas as pl
from jax.experimental.pallas import tpu as pltpu
from jax.experimental.pallas import tpu_sc as plsc

N = 10000
D_IN = 128
HEADS = 8
HID = 16
DH = HEADS * HID
DHALF = DH // 2
D_OUT = 128

NP_ = 10112
NC = 2
NS = 16
NW = NC * NS
K = 128
ZR = NP_ // NS

_mesh = plsc.VectorSubcoreMesh(
    core_axis_name="c", subcore_axis_name="s", num_cores=NC, num_subcores=NS
)
_sc_params = pltpu.CompilerParams(use_tc_tiling_on_sc=False)

f32 = jnp.float32
i32 = jnp.int32


def _k0_body(x_ref, wg_ref, a1_ref, a2_ref, hh_ref, als_ref, ald_ref, m_ref):
    h = jnp.dot(x_ref[...], wg_ref[...], preferred_element_type=f32)
    hh_ref[0] = h[:, :DHALF]
    hh_ref[1] = h[:, DHALF:]
    als = jnp.dot(h, a1_ref[...], preferred_element_type=f32)
    ald = jnp.dot(h, a2_ref[...], preferred_element_type=f32)
    als_ref[...] = als
    ald_ref[...] = ald
    s = jnp.max(als, axis=0, keepdims=True) + jnp.max(ald, axis=0, keepdims=True)
    mlr = jnp.maximum(s, 0.2 * s)
    col = lax.broadcasted_iota(i32, (1, 16), 1)
    m_ref[...] = jnp.where(col < HEADS, mlr, 1e9)


def _kd_body(acc_ref, den_ref, wgcn_ref, r1_ref, r2_ref, out_ref):
    acc = jnp.concatenate([acc_ref[0], acc_ref[1]], axis=1)
    den_all = den_ref[0] + den_ref[1]
    den128 = jnp.dot(den_all, r1_ref[...], preferred_element_type=f32)
    deg128 = jnp.dot(den_all, r2_ref[...], preferred_element_type=f32)
    ygat = acc / (den128 + 1e-16)
    dinv = jnp.where(deg128 > 0, lax.rsqrt(deg128), 0.0)
    h2p = dinv * jnp.dot(ygat, wgcn_ref[...], preferred_element_type=f32)
    out_ref[0] = h2p[:, :DHALF]
    out_ref[1] = h2p[:, DHALF:]


def _ke_body(acc_ref, den_ref, r2_ref, out_ref):
    den_all = den_ref[0] + den_ref[1]
    deg128 = jnp.dot(den_all, r2_ref[...], preferred_element_type=f32)
    dinv = jnp.where(deg128 > 0, lax.rsqrt(deg128), 0.0)
    out_ref[...] = dinv * jnp.concatenate([acc_ref[0], acc_ref[1]], axis=1)


def _make_attn_kernel(chw):
    @functools.partial(
        pl.kernel,
        out_type=(
            jax.ShapeDtypeStruct((NW * chw, K, 16), f32),
            jax.ShapeDtypeStruct((NC, NP_, 16), f32),
        ),
        mesh=_mesh,
        compiler_params=_sc_params,
        scratch_types=(
            pltpu.VMEM((2, K), i32),
            pltpu.VMEM((2, K), i32),
            pltpu.VMEM((K, 16), f32),
            pltpu.VMEM((K, 16), f32),
            pltpu.VMEM((K, 16), f32),
            pltpu.VMEM((K, 16), f32),
            pltpu.VMEM((K, 16), f32),
            pltpu.VMEM((K, 16), f32),
            pltpu.VMEM((1, 16), f32),
            pltpu.VMEM((ZR, 16), f32),
            pltpu.SemaphoreType.DMA,
            pltpu.SemaphoreType.DMA,
            pltpu.SemaphoreType.DMA,
            pltpu.SemaphoreType.DMA,
            pltpu.SemaphoreType.DMA,
            pltpu.SemaphoreType.DMA,
            pltpu.SemaphoreType.DMA,
            pltpu.SemaphoreType.DMA,
            pltpu.VMEM_SHARED((NP_, 16), f32),
        ),
    )
    def attn(als_hbm, ald_hbm, m_hbm, sd_hbm, ex_hbm, den_hbm,
             ix2a, ix2b, g1a, g1b, g2a, g2b, exa, exb2, mv, zb,
             sg1a, sg1b, sg2a, sg2b, sexa, sexb, ssca, sscb, den_sh):
        c = lax.axis_index("c")
        s = lax.axis_index("s")
        wid = s * NC + c
        ix2 = (ix2a, ix2b)
        g1 = (g1a, g1b)
        g2 = (g2a, g2b)
        exv = (exa, exb2)
        sg1 = (sg1a, sg1b)
        sg2 = (sg2a, sg2b)
        sex = (sexa, sexb)
        ssc = (ssca, sscb)
        zvec = jnp.zeros((16,), f32)

        def zrow(i, _):
            zb[i] = zvec
            return 0

        lax.fori_loop(0, ZR, zrow, 0)
        pltpu.sync_copy(zb, den_sh.at[pl.ds(s * ZR, ZR)])
        pltpu.sync_copy(m_hbm, mv)
        pltpu.sync_copy(sd_hbm.at[wid, 0], ix2[0])
        pltpu.async_copy(als_hbm.at[ix2[0].at[0]], g1[0], sg1[0])
        pltpu.async_copy(ald_hbm.at[ix2[0].at[1]], g2[0], sg2[0])
        plsc.subcore_barrier()

        mvv = mv[0]
        is8 = lax.iota(i32, 16) == HEADS

        def pair(p, _):
            for b in (0, 1):
                ci = 2 * p + b
                nb = 1 - b

                @pl.when(ci >= 1)
                def _():
                    pltpu.make_async_copy(exv[nb], ex_hbm.at[wid * chw + ci], sex[nb]).wait()
                    pltpu.make_async_copy(exv[nb], den_sh.at[ix2[nb].at[1]], ssc[nb]).wait()

                @pl.when(ci + 1 < chw)
                def _():
                    pltpu.sync_copy(sd_hbm.at[wid, ci + 1], ix2[nb])
                    pltpu.async_copy(als_hbm.at[ix2[nb].at[0]], g1[nb], sg1[nb])
                    pltpu.async_copy(ald_hbm.at[ix2[nb].at[1]], g2[nb], sg2[nb])

                pltpu.make_async_copy(als_hbm.at[ix2[b].at[0]], g1[b], sg1[b]).wait()
                pltpu.make_async_copy(ald_hbm.at[ix2[b].at[1]], g2[b], sg2[b]).wait()

                def row(i, _):
                    e = g1[b][i] + g2[b][i]
                    e = jnp.maximum(e, 0.2 * e)
                    ex = jnp.exp(e - mvv)
                    exv[b][i] = jnp.where(is8, 1.0, ex)
                    return 0

                lax.fori_loop(0, K, row, 0)
                pltpu.async_copy(exv[b], ex_hbm.at[wid * chw + ci], sex[b])
                pltpu.async_copy(exv[b], den_sh.at[ix2[b].at[1]], ssc[b], add=True)
            return 0

        lax.fori_loop(0, chw // 2, pair, 0)
        bl = (chw - 1) % 2
        pltpu.make_async_copy(exv[bl], ex_hbm.at[wid * chw + chw - 1], sex[bl]).wait()
        pltpu.make_async_copy(exv[bl], den_sh.at[ix2[bl].at[1]], ssc[bl]).wait()
        plsc.subcore_barrier()
        pltpu.sync_copy(den_sh.at[pl.ds(s * ZR, ZR)], zb)
        pltpu.sync_copy(zb, den_hbm.at[c, pl.ds(s * ZR, ZR)])

    return attn


def _make_agg_kernel(cht, scaled):
    @functools.partial(
        pl.kernel,
        out_type=jax.ShapeDtypeStruct((NC, NP_, DHALF), f32),
        mesh=_mesh,
        compiler_params=_sc_params,
        scratch_types=(
            pltpu.VMEM((2, K), i32),
            pltpu.VMEM((2, K), i32),
            pltpu.VMEM((2, K), i32),
            pltpu.VMEM((2, K), i32),
            pltpu.VMEM((K, DHALF), f32),
            pltpu.VMEM((K, DHALF), f32),
            pltpu.VMEM((K, 16), f32),
            pltpu.VMEM((K, 16), f32),
            pltpu.VMEM((ZR, DHALF), f32),
            pltpu.SemaphoreType.DMA,
            pltpu.SemaphoreType.DMA,
            pltpu.SemaphoreType.DMA,
            pltpu.SemaphoreType.DMA,
            pltpu.SemaphoreType.DMA,
            pltpu.SemaphoreType.DMA,
            pltpu.SemaphoreType.DMA,
            pltpu.SemaphoreType.DMA,
            pltpu.SemaphoreType.DMA,
            pltpu.SemaphoreType.DMA,
            pltpu.VMEM_SHARED((NP_, DHALF), f32),
        ),
    )
    def agg(tab_hbm, ex_hbm, sd_hbm, out_hbm,
            ixA, ixB, ixC, ixD, rowsa, rowsb, exa, exb2, zb,
            sixA, sixB, sixC, sixD, sga, sgb, sea, seb, ssa, ssb, acc_sh):
        c = lax.axis_index("c")
        s = lax.axis_index("s")
        ix2 = (ixA, ixB, ixC, ixD)
        six = (sixA, sixB, sixC, sixD)
        rows = (rowsa, rowsb)
        exv = (exa, exb2)
        sg = (sga, sgb)
        se = (sea, seb)
        ss = (ssa, ssb)
        zvec = jnp.zeros((16,), f32)

        def zrow(i, _):
            for j in range(DHALF // 16):
                zb[i, pl.ds(16 * j, 16)] = zvec
            return 0

        lax.fori_loop(0, ZR, zrow, 0)
        pltpu.sync_copy(zb, acc_sh.at[pl.ds(s * ZR, ZR)])
        pltpu.sync_copy(sd_hbm.at[s, 0], ix2[0])
        pltpu.async_copy(sd_hbm.at[s, 1], ix2[1], six[1])

        def gather(cc, ib, rb):
            def go():
                pltpu.async_copy(tab_hbm.at[cc].at[ix2[ib].at[0]], rows[rb], sg[rb])
            return go

        def gwait(cc, ib, rb):
            def go():
                pltpu.make_async_copy(
                    tab_hbm.at[cc].at[ix2[ib].at[0]], rows[rb], sg[rb]).wait()
            return go

        pl.when(c == 0)(gather(0, 0, 0))
        pl.when(c != 0)(gather(1, 0, 0))
        if scaled:
            pltpu.async_copy(ex_hbm.at[s * cht], exv[0], se[0])
        plsc.subcore_barrier()

        def quad(p4, _):
            for b4 in (0, 1, 2, 3):
                ci = 4 * p4 + b4
                rb = b4 % 2
                prb = 1 - rb
                ib = b4
                nib = (b4 + 1) % 4
                fib = (b4 + 2) % 4

                @pl.when(ci >= 1)
                def _():
                    pltpu.make_async_copy(
                        rows[prb], acc_sh.at[ix2[(b4 + 3) % 4].at[1]], ss[prb]).wait()

                @pl.when(ci + 2 < cht)
                def _():
                    pltpu.async_copy(sd_hbm.at[s, ci + 2], ix2[fib], six[fib])

                @pl.when(ci + 1 < cht)
                def _():
                    pltpu.make_async_copy(sd_hbm.at[s, ci + 1], ix2[nib], six[nib]).wait()
                    pl.when(c == 0)(gather(0, nib, prb))
                    pl.when(c != 0)(gather(1, nib, prb))
                    if scaled:
                        pltpu.async_copy(ex_hbm.at[s * cht + ci + 1], exv[prb], se[prb])

                pl.when(c == 0)(gwait(0, ib, rb))
                pl.when(c != 0)(gwait(1, ib, rb))
                if scaled:
                    pltpu.make_async_copy(ex_hbm.at[s * cht + ci], exv[rb], se[rb]).wait()

                    def scale(off):
                        def run():
                            def row(i, _):
                                exr = exv[rb][i]
                                for j in range(DHALF // 16):
                                    sl = pl.ds(16 * j, 16)
                                    rows[rb][i, sl] = rows[rb][i, sl] * exr[off + j]
                                return 0

                            lax.fori_loop(0, K, row, 0)
                        return run

                    pl.when(c == 0)(scale(0))
                    pl.when(c != 0)(scale(4))
                pltpu.async_copy(rows[rb], acc_sh.at[ix2[ib].at[1]], ss[rb], add=True)
            return 0

        lax.fori_loop(0, cht // 4, quad, 0)
        bl = (cht - 1) % 2
        pltpu.make_async_copy(rows[bl], acc_sh.at[ix2[(cht - 1) % 4].at[1]], ss[bl]).wait()
        plsc.subcore_barrier()
        pltpu.sync_copy(acc_sh.at[pl.ds(s * ZR, ZR)], zb)
        pltpu.sync_copy(zb, out_hbm.at[c, pl.ds(s * ZR, ZR)])

    return agg


def kernel(x, edge_index, W_gat, a_src, a_dst, W_gcn):
    E = edge_index.shape[1]
    E2 = E + N
    EP = -(-E2 // (2 * NW * K)) * (2 * NW * K)
    PAD = EP - E2
    CHW = EP // (NW * K)
    CHT = EP // (NS * K)

    xp = jnp.pad(x, ((0, NP_ - N), (0, 0)))
    eye8 = jnp.eye(HEADS, dtype=f32)
    a1 = jnp.concatenate(
        [(eye8[:, None, :] * a_src[:, :, None]).reshape(DH, HEADS),
         jnp.zeros((DH, HEADS), f32)], axis=1)
    a2 = jnp.concatenate(
        [(eye8[:, None, :] * a_dst[:, :, None]).reshape(DH, HEADS),
         jnp.zeros((DH, HEADS), f32)], axis=1)
    r1 = jnp.concatenate(
        [jnp.kron(eye8, jnp.ones((1, HID), f32)), jnp.zeros((8, DH), f32)],
        axis=0)
    r2 = jnp.zeros((16, DH), f32).at[HEADS].set(1.0)

    loop = jnp.arange(N, dtype=i32)
    padv = jnp.full((PAD,), N, dtype=i32)
    srcp = jnp.concatenate([edge_index[0].astype(i32), loop, padv])
    dstp = jnp.concatenate([edge_index[1].astype(i32), loop, padv])
    sd_attn = jnp.concatenate(
        [srcp.reshape(NW, CHW, 1, K), dstp.reshape(NW, CHW, 1, K)], axis=2)
    sd_agg = jnp.concatenate(
        [srcp.reshape(NS, CHT, 1, K), dstp.reshape(NS, CHT, 1, K)], axis=2)

    hh, als, ald, m16 = pl.pallas_call(
        _k0_body,
        out_shape=(
            jax.ShapeDtypeStruct((NC, NP_, DHALF), f32),
            jax.ShapeDtypeStruct((NP_, 16), f32),
            jax.ShapeDtypeStruct((NP_, 16), f32),
            jax.ShapeDtypeStruct((1, 16), f32),
        ),
    )(xp, W_gat, a1, a2)

    ex3, den2 = _make_attn_kernel(CHW)(als, ald, m16, sd_attn)
    ex_agg = ex3.reshape(NS * CHT, K, 16)
    acc_gat = _make_agg_kernel(CHT, scaled=True)(hh, ex_agg, sd_agg)

    h2h = pl.pallas_call(
        _kd_body,
        out_shape=jax.ShapeDtypeStruct((NC, NP_, DHALF), f32),
    )(acc_gat, den2, W_gcn, r1, r2)

    acc_gcn = _make_agg_kernel(CHT, scaled=False)(h2h, ex_agg, sd_agg)

    y = pl.pallas_call(
        _ke_body,
        out_shape=jax.ShapeDtypeStruct((NP_, D_OUT), f32),
    )(acc_gcn, den2, r2)

    return y[:N]
